# baseline (speedup 1.0000x reference)
import functools

import jax
import jax.numpy as jnp
from jax import lax
from jax.experimental import pallas as pl
from jax.experimental.pallas import tpu as pltpu
from jax.experimental.pallas import tpu_sc as plsc


def _mm_body(x_ref, w_ref, o_ref, x16_ref):
    @pl.when(pl.program_id(0) == 0)
    def _():
        x16_ref[...] = x_ref[...].astype(jnp.bfloat16)

    w_blk = w_ref[...].astype(jnp.bfloat16)
    o_ref[...] = jax.lax.dot_general(
        x16_ref[...], w_blk,
        dimension_numbers=(((1,), (1,)), ((), ())),
        preferred_element_type=jnp.float32)


@functools.partial(jax.jit, static_argnames=("n_blk",))
def _spmm(x, W, n_blk=256):
    m, kdim = x.shape
    ndim = W.shape[0]
    return pl.pallas_call(
        _mm_body,
        grid=(ndim // n_blk,),
        in_specs=[
            pl.BlockSpec((m, kdim), lambda n: (0, 0)),
            pl.BlockSpec((n_blk, kdim), lambda n: (n, 0)),
        ],
        out_specs=pl.BlockSpec((m, n_blk), lambda n: (0, n)),
        out_shape=jax.ShapeDtypeStruct((m, ndim), jnp.float32),
        scratch_shapes=[pltpu.VMEM((m, kdim), jnp.bfloat16)],
        compiler_params=pltpu.CompilerParams(
            dimension_semantics=("arbitrary",)),
    )(x, W)


_CHUNK = 65536
_CHUNKS_PER_TEC = 32


def _sc_stream_body(w_hbm, out_hbm, buf, tok):
    wid = lax.axis_index("s") * 2 + lax.axis_index("c")
    base = wid * (_CHUNK * _CHUNKS_PER_TEC)
    for i in range(_CHUNKS_PER_TEC):
        pltpu.sync_copy(w_hbm.at[pl.ds(base + i * _CHUNK, _CHUNK)], buf)
    tok[...] = buf[pl.ds(0, 16)]
    pltpu.sync_copy(tok, out_hbm.at[wid])


@jax.jit
def _sc_stream(w_flat):
    mesh = plsc.VectorSubcoreMesh(core_axis_name="c", subcore_axis_name="s")
    k = functools.partial(
        pl.kernel,
        out_type=jax.ShapeDtypeStruct((32, 16), jnp.float32),
        mesh=mesh,
        scratch_types=[
            pltpu.VMEM((_CHUNK,), jnp.float32),
            pltpu.VMEM((16,), jnp.float32),
        ],
    )(_sc_stream_body)
    return k(w_flat)


def kernel(x, W, bias):
    del bias
    out_tc = _spmm(x, W)
    beta_words = _CHUNK * _CHUNKS_PER_TEC * 32
    w_tail = W.reshape(-1)[-beta_words:]
    junk = _sc_stream(w_tail)
    return out_tc + junk[0, 0] * 0.0


# speedup vs baseline: 3.0484x; 3.0484x over previous
import functools

import jax
import jax.numpy as jnp
from jax import lax
from jax.experimental import pallas as pl
from jax.experimental.pallas import tpu as pltpu
from jax.experimental.pallas import tpu_sc as plsc


def _mm_body(x_ref, w_ref, o_ref, x16_ref):
    @pl.when(pl.program_id(0) == 0)
    def _():
        x16_ref[...] = x_ref[...].astype(jnp.bfloat16)

    w_blk = w_ref[...].astype(jnp.bfloat16)
    o_ref[...] = jax.lax.dot_general(
        x16_ref[...], w_blk,
        dimension_numbers=(((1,), (1,)), ((), ())),
        preferred_element_type=jnp.float32)


@functools.partial(jax.jit, static_argnames=("n_blk",))
def _spmm(x, W, n_blk=256):
    m, kdim = x.shape
    ndim = W.shape[0]
    return pl.pallas_call(
        _mm_body,
        grid=(ndim // n_blk,),
        in_specs=[
            pl.BlockSpec((m, kdim), lambda n: (0, 0)),
            pl.BlockSpec((n_blk, kdim), lambda n: (n, 0)),
        ],
        out_specs=pl.BlockSpec((m, n_blk), lambda n: (0, n)),
        out_shape=jax.ShapeDtypeStruct((m, ndim), jnp.float32),
        scratch_shapes=[pltpu.VMEM((m, kdim), jnp.bfloat16)],
        compiler_params=pltpu.CompilerParams(
            dimension_semantics=("arbitrary",)),
    )(x, W)


_ROWS_PER_DMA = 4
_DMAS_PER_TEC = 32
_ROW0 = 12288


def _sc_stream_body(w_hbm, out_hbm, buf, tok):
    wid = lax.axis_index("s") * 2 + lax.axis_index("c")
    base = _ROW0 + wid * (_ROWS_PER_DMA * _DMAS_PER_TEC)
    for i in range(_DMAS_PER_TEC):
        pltpu.sync_copy(
            w_hbm.at[pl.ds(base + i * _ROWS_PER_DMA, _ROWS_PER_DMA)], buf)
    tok[...] = buf[0, pl.ds(0, 16)]
    pltpu.sync_copy(tok, out_hbm.at[wid])


@jax.jit
def _sc_stream(W):
    mesh = plsc.VectorSubcoreMesh(core_axis_name="c", subcore_axis_name="s")
    k = functools.partial(
        pl.kernel,
        out_type=jax.ShapeDtypeStruct((32, 16), jnp.float32),
        mesh=mesh,
        scratch_types=[
            pltpu.VMEM((_ROWS_PER_DMA, 16384), jnp.float32),
            pltpu.VMEM((16,), jnp.float32),
        ],
    )(_sc_stream_body)
    return k(W)


def kernel(x, W, bias):
    del bias
    out_tc = _spmm(x, W)
    junk = _sc_stream(W)
    return out_tc + junk[0, 0] * 0.0


# speedup vs baseline: 3.4925x; 1.1457x over previous
import functools

import jax
import jax.numpy as jnp
from jax.experimental import pallas as pl
from jax.experimental.pallas import tpu as pltpu


def _mm_body(x_ref, w_ref, o_ref, x16_ref):
    @pl.when(pl.program_id(0) == 0)
    def _():
        x16_ref[...] = x_ref[...].astype(jnp.bfloat16)

    w_blk = w_ref[...].astype(jnp.bfloat16)
    o_ref[...] = jax.lax.dot_general(
        x16_ref[...], w_blk,
        dimension_numbers=(((1,), (1,)), ((), ())),
        preferred_element_type=jnp.float32)


@functools.partial(jax.jit, static_argnames=("n_blk",))
def _spmm(x, W, n_blk=128):
    m, kdim = x.shape
    ndim = W.shape[0]
    return pl.pallas_call(
        _mm_body,
        grid=(ndim // n_blk,),
        in_specs=[
            pl.BlockSpec((m, kdim), lambda n: (0, 0)),
            pl.BlockSpec((n_blk, kdim), lambda n: (n, 0)),
        ],
        out_specs=pl.BlockSpec((m, n_blk), lambda n: (0, n)),
        out_shape=jax.ShapeDtypeStruct((m, ndim), jnp.float32),
        scratch_shapes=[pltpu.VMEM((m, kdim), jnp.bfloat16)],
        compiler_params=pltpu.CompilerParams(
            dimension_semantics=("arbitrary",)),
    )(x, W)


def kernel(x, W, bias):
    del bias
    return _spmm(x, W)


# speedup vs baseline: 4.0680x; 1.1648x over previous
import functools

import jax
import jax.numpy as jnp
from jax.experimental import pallas as pl
from jax.experimental.pallas import tpu as pltpu


def _mm_body(x_ref, w_ref, o_ref, x16_ref):
    @pl.when(pl.program_id(0) == 0)
    def _():
        x16_ref[...] = x_ref[...].astype(jnp.bfloat16)

    w_blk = w_ref[...].astype(jnp.bfloat16)
    o_ref[...] = jax.lax.dot_general(
        x16_ref[...], w_blk,
        dimension_numbers=(((1,), (1,)), ((), ())),
        preferred_element_type=jnp.float32)


@functools.partial(jax.jit, static_argnames=("n_blk",))
def _spmm(x, W, n_blk=256):
    m, kdim = x.shape
    ndim = W.shape[0]
    return pl.pallas_call(
        _mm_body,
        grid=(ndim // n_blk,),
        in_specs=[
            pl.BlockSpec((m, kdim), lambda n: (0, 0)),
            pl.BlockSpec((n_blk, kdim), lambda n: (n, 0)),
        ],
        out_specs=pl.BlockSpec((m, n_blk), lambda n: (0, n)),
        out_shape=jax.ShapeDtypeStruct((m, ndim), jnp.float32),
        scratch_shapes=[pltpu.VMEM((m, kdim), jnp.bfloat16)],
        compiler_params=pltpu.CompilerParams(
            dimension_semantics=("arbitrary",)),
    )(x, W)


def kernel(x, W, bias):
    del bias
    return _spmm(x, W)
